# baseline (speedup 1.0000x reference)
import functools

import jax
import jax.numpy as jnp
from jax import lax
from jax.experimental import pallas as pl
from jax.experimental.pallas import tpu as pltpu
from jax.experimental.pallas import tpu_sc as plsc

_NUM_BUCKETS = 32
_BUCKET_SIZE = 10
_DIM = 128
_BATCH = 16384

_NC = 2
_NS = 16
_L = 16
_NW = _NC * _NS
_BPW = _BATCH // _NW

_mesh = plsc.VectorSubcoreMesh(core_axis_name="c", subcore_axis_name="s")


@functools.partial(
    pl.kernel,
    out_type=jax.ShapeDtypeStruct((_BATCH, _DIM), jnp.float32),
    mesh=_mesh,
    scratch_types=[
        pltpu.VMEM((_BPW,), jnp.int32),
        pltpu.VMEM((_BPW, _DIM), jnp.float32),
        pltpu.SemaphoreType.DMA,
    ],
)
def _lookup(lengths_hbm, table_hbm, out_hbm, idx_v, rows_v, sem):
    wid = lax.axis_index("s") * _NC + lax.axis_index("c")
    base = wid * _BPW

    pltpu.sync_copy(lengths_hbm.at[pl.ds(base, _BPW)], idx_v)

    def body(i, carry):
        x = idx_v[pl.ds(i * _L, _L)]
        b = jnp.minimum((x * 6554) >> 16, _NUM_BUCKETS - 1)
        idx_v[pl.ds(i * _L, _L)] = b
        return carry

    lax.fori_loop(0, _BPW // _L, body, 0)

    pltpu.async_copy(table_hbm.at[idx_v], rows_v, sem).wait()
    pltpu.sync_copy(rows_v, out_hbm.at[pl.ds(base, _BPW)])


def kernel(lengths, embedding):
    out = _lookup(lengths.astype(jnp.int32), embedding)
    return out[:, None, :]


# speedup vs baseline: 2.7225x; 2.7225x over previous
import functools

import jax
import jax.numpy as jnp
from jax import lax
from jax.experimental import pallas as pl
from jax.experimental.pallas import tpu as pltpu
from jax.experimental.pallas import tpu_sc as plsc

_NUM_BUCKETS = 32
_BUCKET_SIZE = 10
_DIM = 128
_BATCH = 16384

_NC = 2
_NS = 16
_L = 16
_NW = _NC * _NS
_BPW = _BATCH // _NW

_mesh = plsc.VectorSubcoreMesh(core_axis_name="c", subcore_axis_name="s")


@functools.partial(
    pl.kernel,
    out_type=jax.ShapeDtypeStruct((_BATCH * _DIM,), jnp.float32),
    mesh=_mesh,
    compiler_params=pltpu.CompilerParams(needs_layout_passes=False),
    scratch_types=[
        pltpu.VMEM((_BPW,), jnp.int32),
        pltpu.VMEM((_NUM_BUCKETS * _DIM,), jnp.float32),
        pltpu.VMEM((_BPW * _DIM,), jnp.float32),
        pltpu.SemaphoreType.DMA,
    ],
)
def _lookup(lengths_hbm, table_hbm, out_hbm, idx_v, table_v, rows_v, sem):
    wid = lax.axis_index("s") * _NC + lax.axis_index("c")
    base = wid * _BPW

    tab_cp = pltpu.async_copy(table_hbm, table_v, sem)
    pltpu.sync_copy(lengths_hbm.at[pl.ds(base, _BPW)], idx_v)
    tab_cp.wait()

    def gbody(g, carry):
        x = idx_v[pl.ds(g * _L, _L)]
        b = jnp.minimum((x * 6554) >> 16, _NUM_BUCKETS - 1)
        src = b * _DIM
        dst = (lax.iota(jnp.int32, _L) + g * _L) * _DIM

        def cbody(c, carry2):
            vals = plsc.load_gather(table_v, [src + c])
            plsc.store_scatter(rows_v, [dst + c], vals)
            return carry2

        lax.fori_loop(0, _DIM, cbody, 0, unroll=8)
        return carry

    lax.fori_loop(0, _BPW // _L, gbody, 0)

    pltpu.sync_copy(rows_v, out_hbm.at[pl.ds(base * _DIM, _BPW * _DIM)])


def kernel(lengths, embedding):
    out = _lookup(lengths.astype(jnp.int32), embedding.reshape(-1))
    return out.reshape(_BATCH, 1, _DIM)


# speedup vs baseline: 10.2400x; 3.7613x over previous
import functools

import jax
import jax.numpy as jnp
from jax import lax
from jax.experimental import pallas as pl
from jax.experimental.pallas import tpu as pltpu
from jax.experimental.pallas import tpu_sc as plsc

_NUM_BUCKETS = 32
_BUCKET_SIZE = 10
_DIM = 128
_BATCH = 16384

_NC = 2
_NS = 16
_L = 16
_NW = _NC * _NS
_BPW = _BATCH // _NW

_mesh = plsc.VectorSubcoreMesh(core_axis_name="c", subcore_axis_name="s")


@functools.partial(
    pl.kernel,
    out_type=jax.ShapeDtypeStruct((_BATCH, _DIM), jnp.float32),
    mesh=_mesh,
    compiler_params=pltpu.CompilerParams(needs_layout_passes=False),
    scratch_types=[
        pltpu.VMEM((_BPW,), jnp.int32),
        pltpu.VMEM_SHARED((_NUM_BUCKETS, _DIM), jnp.float32),
        pltpu.VMEM((_BPW, _DIM), jnp.float32),
        pltpu.SemaphoreType.DMA,
    ],
)
def _lookup(lengths_hbm, table_hbm, out_hbm, idx_v, table_sh, rows_v, sem):
    sid = lax.axis_index("s")
    wid = sid * _NC + lax.axis_index("c")
    base = wid * _BPW

    @pl.when(sid == 0)
    def _stage_table():
        pltpu.sync_copy(table_hbm, table_sh)

    pltpu.sync_copy(lengths_hbm.at[pl.ds(base, _BPW)], idx_v)

    def body(i, carry):
        x = idx_v[pl.ds(i * _L, _L)]
        b = jnp.minimum((x * 6554) >> 16, _NUM_BUCKETS - 1)
        idx_v[pl.ds(i * _L, _L)] = b
        return carry

    lax.fori_loop(0, _BPW // _L, body, 0)
    plsc.subcore_barrier()

    pltpu.async_copy(table_sh.at[idx_v], rows_v, sem).wait()
    pltpu.sync_copy(rows_v, out_hbm.at[pl.ds(base, _BPW)])


def kernel(lengths, embedding):
    out = _lookup(lengths.astype(jnp.int32), embedding)
    return out[:, None, :]


# speedup vs baseline: 10.4937x; 1.0248x over previous
import functools

import jax
import jax.numpy as jnp
from jax import lax
from jax.experimental import pallas as pl
from jax.experimental.pallas import tpu as pltpu
from jax.experimental.pallas import tpu_sc as plsc

_NUM_BUCKETS = 32
_BUCKET_SIZE = 10
_DIM = 128
_BATCH = 16384

_NC = 2
_NS = 16
_L = 16
_NW = _NC * _NS
_BPW = _BATCH // _NW

_mesh = plsc.VectorSubcoreMesh(core_axis_name="c", subcore_axis_name="s")


@functools.partial(
    pl.kernel,
    out_type=jax.ShapeDtypeStruct((_BATCH, _DIM), jnp.float32),
    mesh=_mesh,
    compiler_params=pltpu.CompilerParams(needs_layout_passes=False),
    scratch_types=[
        pltpu.VMEM((_BPW,), jnp.int32),
        pltpu.VMEM_SHARED((_NUM_BUCKETS, _DIM), jnp.float32),
        pltpu.VMEM((_BPW, _DIM), jnp.float32),
        pltpu.SemaphoreType.DMA,
        pltpu.SemaphoreType.DMA,
    ],
)
def _lookup(lengths_hbm, table_hbm, out_hbm, idx_v, table_sh, rows_v, sem_g, sem_w):
    sid = lax.axis_index("s")
    wid = sid * _NC + lax.axis_index("c")
    base = wid * _BPW

    @pl.when(sid == 0)
    def _stage_table():
        pltpu.sync_copy(table_hbm, table_sh)

    pltpu.sync_copy(lengths_hbm.at[pl.ds(base, _BPW)], idx_v)

    def body(i, carry):
        x = idx_v[pl.ds(i * _L, _L)]
        b = jnp.minimum((x * 6554) >> 16, _NUM_BUCKETS - 1)
        idx_v[pl.ds(i * _L, _L)] = b
        return carry

    lax.fori_loop(0, _BPW // _L, body, 0)
    plsc.subcore_barrier()

    n_ch = 4
    ch = _BPW // n_ch

    def gather(k):
        return pltpu.async_copy(
            table_sh.at[idx_v.at[pl.ds(k * ch, ch)]],
            rows_v.at[pl.ds(k * ch, ch)],
            sem_g,
        )

    gathers = [gather(0)]
    writes = []
    for k in range(n_ch):
        gathers[k].wait()
        if k + 1 < n_ch:
            gathers.append(gather(k + 1))
        writes.append(
            pltpu.async_copy(
                rows_v.at[pl.ds(k * ch, ch)],
                out_hbm.at[pl.ds(base + k * ch, ch)],
                sem_w,
            )
        )
    for w in writes:
        w.wait()


def kernel(lengths, embedding):
    out = _lookup(lengths.astype(jnp.int32), embedding)
    return out[:, None, :]
